# Initial kernel scaffold; baseline (speedup 1.0000x reference)
import jax
import jax.numpy as jnp
from jax.experimental import pallas as pl


def kernel(proposal, xyz_feat):
    raise NotImplementedError("write your pallas kernel here")


# speedup vs baseline: 1.1247x; 1.1247x over previous
import jax
import jax.numpy as jnp
from jax import lax
from jax.experimental import pallas as pl
from jax.experimental.pallas import tpu as pltpu
from jax.experimental.pallas import tpu_sc as plsc

NPROP = 32
NPTS = 1024
NVOX = 1728
CAP = 8
ROWS_P = NVOX * CAP
NCHUNK = NPTS // 16
MROW = ROWS_P // 128
NB = 6
LA = 3
NCAND = 27


def _bf16v(v):
    u = plsc.bitcast(v, jnp.uint32)
    r = (u + jnp.uint32(0x7FFF) + ((u >> jnp.uint32(16)) &
                                   jnp.uint32(1))) & jnp.uint32(0xFFFF0000)
    return plsc.bitcast(r, jnp.float32)


def _two_sum(a, b):
    s = a + b
    bp = s - a
    return s, (a - (s - bp)) + (b - bp)


def _sum3(p0, p1, p2):
    s1, e1 = _two_sum(p0, p1)
    s2, e2 = _two_sum(s1, p2)
    return s2 + (e1 + e2)


def _stage_a_body(prm_hbm, xyzt_hbm, vc_hbm, d2_hbm, base_hbm,
                  prm_v, xyzt_v, vc_v, loc_v, d2_v, base_v, asem):
    p = lax.axis_index("c") * 16 + lax.axis_index("s")

    pltpu.sync_copy(prm_hbm.at[p], prm_v)
    pltpu.sync_copy(xyzt_hbm.at[p], xyzt_v)
    pltpu.sync_copy(vc_hbm.at[p], vc_v)

    prm = prm_v[...]
    sy = jnp.full((16,), prm[5], jnp.float32) / 12.0
    sz = jnp.full((16,), prm[6], jnp.float32) / 12.0
    sx = jnp.full((16,), prm[7], jnp.float32) / 12.0

    fsplat = lambda i: jnp.full((16,), prm[i], jnp.float32)
    cb = _bf16v(fsplat(3))
    sb = _bf16v(fsplat(4))
    msb = -sb
    r0b = _bf16v(fsplat(0))
    r1b = _bf16v(fsplat(1))
    r2b = _bf16v(fsplat(2))
    txb = _bf16v(-(cb * r0b + msb * r2b))
    tyb = _bf16v(-r1b)
    tzb = _bf16v(-(sb * r0b + cb * r2b))

    inf = jnp.full((16,), jnp.inf, jnp.float32)

    @pl.loop(0, NCHUNK, init_carry=(inf, -inf, inf, -inf, inf, -inf))
    def _pass1(c, carry):
        mnx, mxx, mny, mxy, mnz, mxz = carry
        b = c * 16
        xb = _bf16v(xyzt_v[0, pl.ds(b, 16)])
        yb = _bf16v(xyzt_v[1, pl.ds(b, 16)])
        zb = _bf16v(xyzt_v[2, pl.ds(b, 16)])
        xl = _sum3(cb * xb, msb * zb, txb)
        yl = yb + tyb
        zl = _sum3(sb * xb, cb * zb, tzb)
        loc_v[0, pl.ds(b, 16)] = xl
        loc_v[1, pl.ds(b, 16)] = yl
        loc_v[2, pl.ds(b, 16)] = zl
        return (jnp.minimum(mnx, xl), jnp.maximum(mxx, xl),
                jnp.minimum(mny, yl), jnp.maximum(mxy, yl),
                jnp.minimum(mnz, zl), jnp.maximum(mxz, zl))

    mnx, mxx, mny, mxy, mnz, mxz = _pass1
    splat = lambda v: jnp.full((16,), v, jnp.float32)
    xminf = splat(jnp.min(mnx)) - fsplat(8)
    xmaxf = splat(jnp.max(mxx)) + fsplat(8)
    yminf = splat(jnp.min(mny)) - fsplat(9)
    ymaxf = splat(jnp.max(mxy)) + fsplat(9)
    zminf = splat(jnp.min(mnz)) - fsplat(10)
    zmaxf = splat(jnp.max(mxz)) + fsplat(10)

    def ifloor(t):
        i = t.astype(jnp.int32)
        return i - (i.astype(jnp.float32) > t).astype(jnp.int32)

    lox = jnp.maximum(ifloor(xminf / sx + 6.0), 0)
    hix = jnp.minimum(ifloor(xmaxf / sx + 6.0), 11)
    loy = jnp.maximum(ifloor(-ymaxf / sy - 0.5), 0)
    hiy = jnp.minimum(ifloor(-yminf / sy - 0.5), 11)
    loz = jnp.maximum(ifloor(zminf / sz + 6.0), 0)
    hiz = jnp.minimum(ifloor(zmaxf / sz + 6.0), 11)

    @pl.loop(0, NCHUNK)
    def _pass2(c):
        b = c * 16
        xl = loc_v[0, pl.ds(b, 16)]
        yl = loc_v[1, pl.ds(b, 16)]
        zl = loc_v[2, pl.ds(b, 16)]
        i0 = jnp.clip(ifloor(xl / sx + 6.0), lox, hix)
        j0 = jnp.clip(ifloor(-yl / sy - 0.5), loy, hiy)
        k0 = jnp.clip(ifloor(zl / sz + 6.0), loz, hiz)
        base_v[pl.ds(b, 16)] = i0 * 144 + j0 * 12 + k0
        comp = []
        for axis, (q0, ql, vlo, vhi) in enumerate(
                ((i0, xl, xminf, xmaxf),
                 (j0, yl, yminf, ymaxf),
                 (k0, zl, zminf, zmaxf))):
            per = []
            for d in (-1, 0, 1):
                qr = q0 + d
                qi = jnp.clip(qr, 0, 11)
                vc = plsc.load_gather(vc_v, [jnp.full((16,), axis,
                                                      jnp.int32), qi])
                dq = vc - ql
                ok = (vc > vlo) & (vc < vhi) & (qr >= 0) & (qr <= 11)
                per.append((ok, dq * dq))
            comp.append(per)
        for ci, (vx, dx2) in enumerate(comp[0]):
            for cj, (vy, dy2) in enumerate(comp[1]):
                sxy = dx2 + dy2
                vxy = vx & vy
                for ck, (vz, dz2) in enumerate(comp[2]):
                    d2 = jnp.where(vxy & vz, sxy + dz2, inf)
                    d2_v[ci * 9 + cj * 3 + ck, pl.ds(b, 16)] = d2

    pltpu.sync_copy(base_v, base_hbm.at[p])
    outcopies = [pltpu.async_copy(d2_v.at[cand],
                                  d2_hbm.at[cand, pl.ds(p * NPTS, NPTS)],
                                  asem)
                 for cand in range(NCAND)]
    for oc in outcopies:
        oc.wait()


def _stage_b_body(d2_ref, base_ref, sel_ref):
    best = d2_ref[0]
    bi = jnp.zeros(best.shape, jnp.int32)
    bj = jnp.zeros(best.shape, jnp.int32)
    bk = jnp.zeros(best.shape, jnp.int32)
    for cand in range(1, NCAND):
        d = d2_ref[cand]
        ds = jnp.sqrt(d)
        bs = jnp.sqrt(best)
        upd = ds < bs
        best = jnp.where(upd, d, best)
        bi = jnp.where(upd, cand // 9, bi)
        bj = jnp.where(upd, (cand // 3) % 3, bj)
        bk = jnp.where(upd, cand % 3, bk)
    sel = (base_ref[...] + (bi - 1) * 144 + (bj - 1) * 12 + (bk - 1))
    sel_ref[...] = jnp.where(jnp.isinf(best), 0, sel)


def _stage_c_body(sel_hbm, vals_hbm, out_hbm,
                  sel_v, map_v, cnt_v, *stages_and_sems):
    stage = stages_and_sems[:NB]
    gsems = stages_and_sems[NB:2 * NB]
    wsems = stages_and_sems[2 * NB:3 * NB]
    p = lax.axis_index("c") * 16 + lax.axis_index("s")

    pltpu.sync_copy(sel_hbm.at[p], sel_v)

    lane = lax.iota(jnp.int32, 16)
    sentinel = NPROP * NPTS + (lane & 7)

    @pl.loop(0, MROW)
    def _fillmap(j):
        for k in range(8):
            map_v[j, pl.ds(k * 16, 16)] = sentinel

    @pl.loop(0, NVOX // 16)
    def _zcnt(r):
        cnt_v[pl.ds(r * 16, 16)] = jnp.zeros((16,), jnp.int32)

    @pl.loop(0, NCHUNK)
    def _pass3(c):
        b = c * 16
        sel = sel_v[pl.ds(b, 16)]
        old = plsc.load_gather(cnt_v, [sel])
        pc = jnp.zeros((16,), jnp.int32)
        nlater = jnp.zeros((16,), jnp.int32)
        for m in range(16):
            eq = sel == sel[m]
            pc = pc + jnp.where(eq & (lane > m), 1, 0)
            nlater = nlater + jnp.where(eq & (lane < m), 1, 0)
        slot = old + pc
        plsc.store_scatter(cnt_v, [sel], slot + 1, mask=nlater == 0)
        row = sel * CAP + slot
        plsc.store_scatter(map_v, [row >> 7, row & 127],
                           p * NPTS + b + lane, mask=slot < CAP)

    def fire_gather(j):
        s = j % NB
        return pltpu.async_copy(
            vals_hbm.at[map_v.at[j]], stage[s], gsems[s])

    gh = [None] * MROW
    wh = [None] * MROW
    for j in range(LA):
        gh[j] = fire_gather(j)
    for j in range(MROW):
        f = j + LA
        if f < MROW:
            if f >= NB:
                wh[f - NB].wait()
            gh[f] = fire_gather(f)
        gh[j].wait()
        wh[j] = pltpu.async_copy(
            stage[j % NB],
            out_hbm.at[pl.ds((p * MROW + j) * 128, 128), :],
            wsems[j % NB])
    for j in range(MROW - NB, MROW):
        wh[j].wait()


@jax.jit
def _voxelize(params, xyzt, vc, vals_pad):
    mesh = plsc.VectorSubcoreMesh(core_axis_name="c", subcore_axis_name="s")
    sc_params = pltpu.CompilerParams(
        needs_layout_passes=False, use_tc_tiling_on_sc=False)

    stage_a = pl.kernel(
        _stage_a_body,
        out_type=(
            jax.ShapeDtypeStruct((NCAND, NPROP * NPTS), jnp.float32),
            jax.ShapeDtypeStruct((NPROP, NPTS), jnp.int32),
        ),
        mesh=mesh,
        scratch_types=[
            pltpu.VMEM((16,), jnp.float32),
            pltpu.VMEM((3, NPTS), jnp.float32),
            pltpu.VMEM((3, 12), jnp.float32),
            pltpu.VMEM((3, NPTS), jnp.float32),
            pltpu.VMEM((NCAND, NPTS), jnp.float32),
            pltpu.VMEM((NPTS,), jnp.int32),
            pltpu.SemaphoreType.DMA,
        ],
        compiler_params=sc_params,
    )
    d2, base = stage_a(params, xyzt, vc)

    sel = pl.pallas_call(
        _stage_b_body,
        out_shape=jax.ShapeDtypeStruct((NPROP * NPTS // 128, 128), jnp.int32),
    )(d2.reshape(NCAND, NPROP * NPTS // 128, 128),
      base.reshape(NPROP * NPTS // 128, 128))

    stage_c = pl.kernel(
        _stage_c_body,
        out_type=jax.ShapeDtypeStruct((NPROP * ROWS_P, 16), jnp.float32),
        mesh=mesh,
        scratch_types=[
            pltpu.VMEM((NPTS,), jnp.int32),
            pltpu.VMEM((MROW, 128), jnp.int32),
            pltpu.VMEM((NVOX,), jnp.int32),
        ] + [pltpu.VMEM((128, 16), jnp.float32) for _ in range(NB)]
          + [pltpu.SemaphoreType.DMA for _ in range(2 * NB)],
        compiler_params=sc_params,
    )
    out = stage_c(sel.reshape(NPROP, NPTS), vals_pad)
    return out


def kernel(proposal, xyz_feat):
    c = jnp.cos(proposal[:, 6:7])
    s = jnp.sin(proposal[:, 6:7])
    he = proposal[:, 3:4] + 0.5
    we = proposal[:, 4:5] + 0.5
    le = proposal[:, 5:6] + 0.5
    halfx = le / (12 * 2)
    halfy = he / (12 * 2)
    halfz = we / (12 * 2)
    params = jnp.concatenate(
        [proposal[:, 0:3], c, s, he, we, le, halfx, halfy, halfz,
         jnp.zeros((NPROP, 5), jnp.float32)], axis=1)
    iv = jnp.arange(-6, 6, dtype=jnp.float32)
    jv = jnp.arange(12, dtype=jnp.float32)
    kv = jnp.arange(-6, 6, dtype=jnp.float32)
    vcx = le * iv[None, :] / 12 + le / (12 * 2)
    vcy = -he * jv[None, :] / 12 - he / (12 * 2)
    vcz = we * kv[None, :] / 12 + we / (12 * 2)
    vc = jnp.stack([vcx, vcy, vcz], axis=1)
    xyzt = jnp.transpose(xyz_feat[:, :, :3], (0, 2, 1))
    vals_pad = jnp.concatenate(
        [jnp.concatenate([xyz_feat[:, :, 3:], xyz_feat[:, :, :3]],
                         axis=-1).reshape(NPROP * NPTS, 16),
         jnp.zeros((8, 16), jnp.float32)], axis=0)
    out = _voxelize(params, xyzt, vc, vals_pad)
    return out.reshape(NPROP, NVOX, CAP, 16)


# speedup vs baseline: 8.9575x; 7.9640x over previous
import jax
import jax.numpy as jnp
from jax import lax
from jax.experimental import pallas as pl
from jax.experimental.pallas import tpu as pltpu
from jax.experimental.pallas import tpu_sc as plsc

NPROP = 32
NPTS = 1024
NVOX = 1728
CAP = 8
ROWS_P = NVOX * CAP
NCHUNK = NPTS // 16
MROW = ROWS_P // 128
NB = 6
LA = 3
NCAND = 27


def _bf16v(v):
    u = plsc.bitcast(v, jnp.uint32)
    r = (u + jnp.uint32(0x7FFF) + ((u >> jnp.uint32(16)) &
                                   jnp.uint32(1))) & jnp.uint32(0xFFFF0000)
    return plsc.bitcast(r, jnp.float32)


def _two_sum(a, b):
    s = a + b
    bp = s - a
    return s, (a - (s - bp)) + (b - bp)


def _sum3(p0, p1, p2):
    s1, e1 = _two_sum(p0, p1)
    s2, e2 = _two_sum(s1, p2)
    return s2 + (e1 + e2)


def _stage_a_body(prm_hbm, xyzt_hbm, vc_hbm, d2_hbm, base_hbm,
                  prm_v, xyzt_v, vc_v, loc_v, d2_v, base_v, asem):
    p = lax.axis_index("c") * 16 + lax.axis_index("s")

    pltpu.sync_copy(prm_hbm.at[p], prm_v)
    pltpu.sync_copy(xyzt_hbm.at[p], xyzt_v)
    pltpu.sync_copy(vc_hbm.at[p], vc_v)

    prm = prm_v[...]
    sy = jnp.full((16,), prm[5], jnp.float32) / 12.0
    sz = jnp.full((16,), prm[6], jnp.float32) / 12.0
    sx = jnp.full((16,), prm[7], jnp.float32) / 12.0

    fsplat = lambda i: jnp.full((16,), prm[i], jnp.float32)
    cb = _bf16v(fsplat(3))
    sb = _bf16v(fsplat(4))
    msb = -sb
    r0b = _bf16v(fsplat(0))
    r1b = _bf16v(fsplat(1))
    r2b = _bf16v(fsplat(2))
    txb = _bf16v(-(cb * r0b + msb * r2b))
    tyb = _bf16v(-r1b)
    tzb = _bf16v(-(sb * r0b + cb * r2b))

    inf = jnp.full((16,), jnp.inf, jnp.float32)

    @pl.loop(0, NCHUNK, init_carry=(inf, -inf, inf, -inf, inf, -inf))
    def _pass1(c, carry):
        mnx, mxx, mny, mxy, mnz, mxz = carry
        b = c * 16
        xb = _bf16v(xyzt_v[0, pl.ds(b, 16)])
        yb = _bf16v(xyzt_v[1, pl.ds(b, 16)])
        zb = _bf16v(xyzt_v[2, pl.ds(b, 16)])
        xl = _sum3(cb * xb, msb * zb, txb)
        yl = yb + tyb
        zl = _sum3(sb * xb, cb * zb, tzb)
        loc_v[0, pl.ds(b, 16)] = xl
        loc_v[1, pl.ds(b, 16)] = yl
        loc_v[2, pl.ds(b, 16)] = zl
        return (jnp.minimum(mnx, xl), jnp.maximum(mxx, xl),
                jnp.minimum(mny, yl), jnp.maximum(mxy, yl),
                jnp.minimum(mnz, zl), jnp.maximum(mxz, zl))

    mnx, mxx, mny, mxy, mnz, mxz = _pass1
    splat = lambda v: jnp.full((16,), v, jnp.float32)
    xminf = splat(jnp.min(mnx)) - fsplat(8)
    xmaxf = splat(jnp.max(mxx)) + fsplat(8)
    yminf = splat(jnp.min(mny)) - fsplat(9)
    ymaxf = splat(jnp.max(mxy)) + fsplat(9)
    zminf = splat(jnp.min(mnz)) - fsplat(10)
    zmaxf = splat(jnp.max(mxz)) + fsplat(10)

    def ifloor(t):
        i = t.astype(jnp.int32)
        return i - (i.astype(jnp.float32) > t).astype(jnp.int32)

    lox = jnp.maximum(ifloor(xminf / sx + 6.0), 0)
    hix = jnp.minimum(ifloor(xmaxf / sx + 6.0), 11)
    loy = jnp.maximum(ifloor(-ymaxf / sy - 0.5), 0)
    hiy = jnp.minimum(ifloor(-yminf / sy - 0.5), 11)
    loz = jnp.maximum(ifloor(zminf / sz + 6.0), 0)
    hiz = jnp.minimum(ifloor(zmaxf / sz + 6.0), 11)

    @pl.loop(0, NCHUNK)
    def _pass2(c):
        b = c * 16
        xl = loc_v[0, pl.ds(b, 16)]
        yl = loc_v[1, pl.ds(b, 16)]
        zl = loc_v[2, pl.ds(b, 16)]
        i0 = jnp.clip(ifloor(xl / sx + 6.0), lox, hix)
        j0 = jnp.clip(ifloor(-yl / sy - 0.5), loy, hiy)
        k0 = jnp.clip(ifloor(zl / sz + 6.0), loz, hiz)
        base_v[pl.ds(b, 16)] = i0 * 144 + j0 * 12 + k0
        comp = []
        for axis, (q0, ql, vlo, vhi) in enumerate(
                ((i0, xl, xminf, xmaxf),
                 (j0, yl, yminf, ymaxf),
                 (k0, zl, zminf, zmaxf))):
            per = []
            for d in (-1, 0, 1):
                qr = q0 + d
                qi = jnp.clip(qr, 0, 11)
                vc = plsc.load_gather(vc_v, [jnp.full((16,), axis,
                                                      jnp.int32), qi])
                dq = vc - ql
                ok = (vc > vlo) & (vc < vhi) & (qr >= 0) & (qr <= 11)
                per.append((ok, dq * dq))
            comp.append(per)
        for ci, (vx, dx2) in enumerate(comp[0]):
            for cj, (vy, dy2) in enumerate(comp[1]):
                sxy = dx2 + dy2
                vxy = vx & vy
                for ck, (vz, dz2) in enumerate(comp[2]):
                    d2 = jnp.where(vxy & vz, sxy + dz2, inf)
                    d2_v[ci * 9 + cj * 3 + ck, pl.ds(b, 16)] = d2

    pltpu.sync_copy(base_v, base_hbm.at[p])
    outcopies = [pltpu.async_copy(d2_v.at[cand],
                                  d2_hbm.at[cand, pl.ds(p * NPTS, NPTS)],
                                  asem)
                 for cand in range(NCAND)]
    for oc in outcopies:
        oc.wait()


def _stage_b_body(d2_ref, base_ref, sel_ref):
    best = d2_ref[0]
    bi = jnp.zeros(best.shape, jnp.int32)
    bj = jnp.zeros(best.shape, jnp.int32)
    bk = jnp.zeros(best.shape, jnp.int32)
    for cand in range(1, NCAND):
        d = d2_ref[cand]
        ds = jnp.sqrt(d)
        bs = jnp.sqrt(best)
        upd = ds < bs
        best = jnp.where(upd, d, best)
        bi = jnp.where(upd, cand // 9, bi)
        bj = jnp.where(upd, (cand // 3) % 3, bj)
        bk = jnp.where(upd, cand % 3, bk)
    sel = (base_ref[...] + (bi - 1) * 144 + (bj - 1) * 12 + (bk - 1))
    sel_ref[...] = jnp.where(jnp.isinf(best), 0, sel)


def _stage_c_body(sel_hbm, vals_hbm, out_hbm,
                  sel_v, map_v, cnt_v, *stages_and_sems):
    stage = stages_and_sems[:NB]
    gsems = stages_and_sems[NB:2 * NB]
    wsems = stages_and_sems[2 * NB:3 * NB]
    p = lax.axis_index("c") * 16 + lax.axis_index("s")

    pltpu.sync_copy(sel_hbm.at[p], sel_v)

    lane = lax.iota(jnp.int32, 16)

    @pl.loop(0, MROW)
    def _fillmap(j):
        for k in range(8):
            map_v[j, pl.ds(k * 16, 16)] = (
                NPROP * NPTS + ((j * 128 + k * 16 + lane) & 1023))

    @pl.loop(0, NVOX // 16)
    def _zcnt(r):
        cnt_v[pl.ds(r * 16, 16)] = jnp.zeros((16,), jnp.int32)

    @pl.loop(0, NCHUNK)
    def _pass3(c):
        b = c * 16
        sel = sel_v[pl.ds(b, 16)]
        old = plsc.load_gather(cnt_v, [sel])
        pc = jnp.zeros((16,), jnp.int32)
        nlater = jnp.zeros((16,), jnp.int32)
        for m in range(16):
            eq = sel == sel[m]
            pc = pc + jnp.where(eq & (lane > m), 1, 0)
            nlater = nlater + jnp.where(eq & (lane < m), 1, 0)
        slot = old + pc
        plsc.store_scatter(cnt_v, [sel], slot + 1, mask=nlater == 0)
        row = sel * CAP + slot
        plsc.store_scatter(map_v, [row >> 7, row & 127],
                           p * NPTS + b + lane, mask=slot < CAP)

    def fire_gather(j):
        s = j % NB
        return pltpu.async_copy(
            vals_hbm.at[map_v.at[j]], stage[s], gsems[s])

    gh = [None] * MROW
    wh = [None] * MROW
    for j in range(LA):
        gh[j] = fire_gather(j)
    for j in range(MROW):
        f = j + LA
        if f < MROW:
            if f >= NB:
                wh[f - NB].wait()
            gh[f] = fire_gather(f)
        gh[j].wait()
        wh[j] = pltpu.async_copy(
            stage[j % NB],
            out_hbm.at[pl.ds((p * MROW + j) * 128, 128), :],
            wsems[j % NB])
    for j in range(MROW - NB, MROW):
        wh[j].wait()


@jax.jit
def _voxelize(params, xyzt, vc, vals_pad):
    mesh = plsc.VectorSubcoreMesh(core_axis_name="c", subcore_axis_name="s")
    sc_params = pltpu.CompilerParams(
        needs_layout_passes=False, use_tc_tiling_on_sc=False)

    stage_a = pl.kernel(
        _stage_a_body,
        out_type=(
            jax.ShapeDtypeStruct((NCAND, NPROP * NPTS), jnp.float32),
            jax.ShapeDtypeStruct((NPROP, NPTS), jnp.int32),
        ),
        mesh=mesh,
        scratch_types=[
            pltpu.VMEM((16,), jnp.float32),
            pltpu.VMEM((3, NPTS), jnp.float32),
            pltpu.VMEM((3, 12), jnp.float32),
            pltpu.VMEM((3, NPTS), jnp.float32),
            pltpu.VMEM((NCAND, NPTS), jnp.float32),
            pltpu.VMEM((NPTS,), jnp.int32),
            pltpu.SemaphoreType.DMA,
        ],
        compiler_params=sc_params,
    )
    d2, base = stage_a(params, xyzt, vc)

    sel = pl.pallas_call(
        _stage_b_body,
        out_shape=jax.ShapeDtypeStruct((NPROP * NPTS // 128, 128), jnp.int32),
    )(d2.reshape(NCAND, NPROP * NPTS // 128, 128),
      base.reshape(NPROP * NPTS // 128, 128))

    stage_c = pl.kernel(
        _stage_c_body,
        out_type=jax.ShapeDtypeStruct((NPROP * ROWS_P, 16), jnp.float32),
        mesh=mesh,
        scratch_types=[
            pltpu.VMEM((NPTS,), jnp.int32),
            pltpu.VMEM((MROW, 128), jnp.int32),
            pltpu.VMEM((NVOX,), jnp.int32),
        ] + [pltpu.VMEM((128, 16), jnp.float32) for _ in range(NB)]
          + [pltpu.SemaphoreType.DMA for _ in range(2 * NB)],
        compiler_params=sc_params,
    )
    out = stage_c(sel.reshape(NPROP, NPTS), vals_pad)
    return out


def kernel(proposal, xyz_feat):
    c = jnp.cos(proposal[:, 6:7])
    s = jnp.sin(proposal[:, 6:7])
    he = proposal[:, 3:4] + 0.5
    we = proposal[:, 4:5] + 0.5
    le = proposal[:, 5:6] + 0.5
    halfx = le / (12 * 2)
    halfy = he / (12 * 2)
    halfz = we / (12 * 2)
    params = jnp.concatenate(
        [proposal[:, 0:3], c, s, he, we, le, halfx, halfy, halfz,
         jnp.zeros((NPROP, 5), jnp.float32)], axis=1)
    iv = jnp.arange(-6, 6, dtype=jnp.float32)
    jv = jnp.arange(12, dtype=jnp.float32)
    kv = jnp.arange(-6, 6, dtype=jnp.float32)
    vcx = le * iv[None, :] / 12 + le / (12 * 2)
    vcy = -he * jv[None, :] / 12 - he / (12 * 2)
    vcz = we * kv[None, :] / 12 + we / (12 * 2)
    vc = jnp.stack([vcx, vcy, vcz], axis=1)
    xyzt = jnp.transpose(xyz_feat[:, :, :3], (0, 2, 1))
    vals_pad = jnp.concatenate(
        [jnp.concatenate([xyz_feat[:, :, 3:], xyz_feat[:, :, :3]],
                         axis=-1).reshape(NPROP * NPTS, 16),
         jnp.zeros((1024, 16), jnp.float32)], axis=0)
    out = _voxelize(params, xyzt, vc, vals_pad)
    return out.reshape(NPROP, NVOX, CAP, 16)
